# baseline (speedup 1.0000x reference)
import functools

import jax
import jax.numpy as jnp
from jax import lax
from jax.experimental import pallas as pl
from jax.experimental.pallas import tpu as pltpu
from jax.experimental.pallas import tpu_sc as plsc

_NC = 2
_NS = 16
_CH = 128


@functools.lru_cache(maxsize=None)
def _build_segsum(Np, H, per):
    NW = _NC * _NS
    rpt = Np // _NS
    mesh = plsc.VectorSubcoreMesh(core_axis_name="c", subcore_axis_name="s")

    @functools.partial(
        pl.kernel,
        mesh=mesh,
        out_type=jax.ShapeDtypeStruct((_NC, Np, H), jnp.float32),
        scratch_types=[
            pltpu.VMEM((_CH,), jnp.int32),
            pltpu.VMEM((_CH,), jnp.int32),
            pltpu.VMEM((_CH,), jnp.int32),
            pltpu.VMEM((_CH,), jnp.int32),
            pltpu.VMEM((_CH, H), jnp.float32),
            pltpu.VMEM((_CH, H), jnp.float32),
            pltpu.VMEM_SHARED((Np, H), jnp.float32),
            pltpu.SemaphoreType.DMA,
            pltpu.SemaphoreType.DMA,
        ],
    )
    def segsum(h_hbm, src1_hbm, dst1_hbm, zeros_hbm, out_hbm,
               sidx0, sidx1, didx0, didx1, rows0, rows1, acc, sem0, sem1):
        cid = lax.axis_index("c")
        sid = lax.axis_index("s")
        wid = sid * _NC + cid
        r0 = sid * rpt
        pltpu.sync_copy(zeros_hbm.at[pl.ds(r0, rpt)], acc.at[pl.ds(r0, rpt)])
        c0 = wid * per
        plsc.subcore_barrier()

        def fetch_idx(row, sbuf, dbuf):
            pltpu.sync_copy(src1_hbm.at[pl.ds(row * _CH, _CH)], sbuf)
            pltpu.sync_copy(dst1_hbm.at[pl.ds(row * _CH, _CH)], dbuf)

        def gstart(sbuf, buf, sem):
            pltpu.make_async_copy(h_hbm.at[sbuf], buf, sem).start()

        def gwait(sbuf, buf, sem):
            pltpu.make_async_copy(h_hbm.at[sbuf], buf, sem).wait()

        def scatter(dbuf, buf):
            pltpu.sync_copy(buf, acc.at[dbuf], add=True)

        def body(k, carry):
            j = c0 + k
            fetch_idx(j, sidx0, didx0)
            pltpu.async_copy(h_hbm.at[sidx0], rows0, sem0).wait()
            scatter(didx0, rows0)
            return carry

        lax.fori_loop(0, per, body, 0)

        plsc.subcore_barrier()
        pltpu.sync_copy(acc.at[pl.ds(r0, rpt)],
                        out_hbm.at[cid, pl.ds(r0, rpt)])

    return segsum


@functools.lru_cache(maxsize=None)
def _build_count(Np, per, W=128):
    NW = _NC * _NS
    rpt = Np // _NS
    mesh = plsc.VectorSubcoreMesh(core_axis_name="c", subcore_axis_name="s")

    @functools.partial(
        pl.kernel,
        mesh=mesh,
        out_type=jax.ShapeDtypeStruct((_NC, Np, W), jnp.float32),
        scratch_types=[
            pltpu.VMEM((_CH,), jnp.int32),
            pltpu.VMEM((_CH, W), jnp.float32),
            pltpu.VMEM_SHARED((Np, W), jnp.float32),
        ],
    )
    def count(dst1_hbm, ones_hbm, zerosw_hbm, out_hbm, didx, ones_v, cacc):
        cid = lax.axis_index("c")
        sid = lax.axis_index("s")
        wid = sid * _NC + cid
        r0 = sid * rpt
        pltpu.sync_copy(ones_hbm, ones_v)
        pltpu.sync_copy(zerosw_hbm.at[pl.ds(r0, rpt)], cacc.at[pl.ds(r0, rpt)])
        c0 = wid * per
        plsc.subcore_barrier()

        def body(j, carry):
            pltpu.sync_copy(dst1_hbm.at[pl.ds((c0 + j) * _CH, _CH)], didx)
            pltpu.sync_copy(ones_v, cacc.at[didx], add=True)
            return carry

        lax.fori_loop(0, per, body, 0)

        plsc.subcore_barrier()
        pltpu.sync_copy(cacc.at[pl.ds(r0, rpt)],
                        out_hbm.at[cid, pl.ds(r0, rpt)])

    return count


def _fuse(p, h, cnt, Wl_i, Wr_i, b_i, relu, nb=8):
    Np, H = h.shape
    BR = Np // nb

    def body(p_ref, h_ref, cnt_ref, wl_ref, wr_ref, b_ref, o_ref):
        inv = 1.0 / jnp.maximum(cnt_ref[...], 1.0)
        agg = (p_ref[0] + p_ref[1]) * inv
        acc = jnp.dot(agg, wl_ref[...], preferred_element_type=jnp.float32)
        acc = acc + jnp.dot(h_ref[...], wr_ref[...],
                            preferred_element_type=jnp.float32)
        acc = acc + b_ref[...]
        if relu:
            acc = jnp.maximum(acc, 0.0)
        o_ref[...] = acc

    return pl.pallas_call(
        body,
        grid=(nb,),
        in_specs=[
            pl.BlockSpec((2, BR, H), lambda i: (0, i, 0)),
            pl.BlockSpec((BR, H), lambda i: (i, 0)),
            pl.BlockSpec((BR, 1), lambda i: (i, 0)),
            pl.BlockSpec((H, H), lambda i: (0, 0)),
            pl.BlockSpec((H, H), lambda i: (0, 0)),
            pl.BlockSpec((1, H), lambda i: (0, 0)),
        ],
        out_specs=pl.BlockSpec((BR, H), lambda i: (i, 0)),
        out_shape=jax.ShapeDtypeStruct((Np, H), jnp.float32),
    )(p, h, cnt, Wl_i, Wr_i, b_i.reshape(1, H))


def kernel(x, edge_index, Wl, Wr, b):
    N, D = x.shape
    E = edge_index.shape[1]
    L = Wl.shape[0]
    NW = _NC * _NS

    Np = (N // 128 + 1) * 128
    nchunks = -(-E // _CH)
    per = -(-nchunks // NW)
    per = per + (per % 2)
    Ep = per * NW * _CH

    pad_dst = N + jnp.arange(Ep, dtype=jnp.int32) % (Np - N)
    src = jnp.full((Ep,), N, jnp.int32).at[:E].set(edge_index[0])
    dst = pad_dst.at[:E].set(edge_index[1])
    hp = jnp.zeros((Np, D), jnp.float32).at[:N].set(x)
    zeros = jnp.zeros((Np, D), jnp.float32)
    onesw = jnp.ones((_CH, D), jnp.float32)

    cparts = _build_count(Np, per, D)(dst, onesw, zeros)
    cnt = (cparts[0, :, :1] + cparts[1, :, :1])

    segsum = _build_segsum(Np, D, per)
    h = hp
    for i in range(L):
        p = segsum(h, src, dst, zeros)
        h = _fuse(p, h, cnt, Wl[i], Wr[i], b[i], relu=(i < L - 1))
    return h[:N]


# speedup vs baseline: 2.2288x; 2.2288x over previous
import functools

import jax
import jax.numpy as jnp
from jax import lax
from jax.experimental import pallas as pl
from jax.experimental.pallas import tpu as pltpu
from jax.experimental.pallas import tpu_sc as plsc

_NC = 2
_NS = 16
_CH = 128


@functools.lru_cache(maxsize=None)
def _build_segsum(Np, H, per):
    NW = _NC * _NS
    rpt = Np // _NS
    mesh = plsc.VectorSubcoreMesh(core_axis_name="c", subcore_axis_name="s")

    @functools.partial(
        pl.kernel,
        mesh=mesh,
        out_type=jax.ShapeDtypeStruct((_NC, Np, H), jnp.float32),
        scratch_types=[
            pltpu.VMEM((_CH,), jnp.int32),
            pltpu.VMEM((_CH,), jnp.int32),
            pltpu.VMEM((_CH,), jnp.int32),
            pltpu.VMEM((_CH,), jnp.int32),
            pltpu.VMEM((_CH, H), jnp.float32),
            pltpu.VMEM((_CH, H), jnp.float32),
            pltpu.VMEM_SHARED((Np, H), jnp.float32),
            pltpu.SemaphoreType.DMA,
            pltpu.SemaphoreType.DMA,
        ],
    )
    def segsum(h_hbm, src1_hbm, dst1_hbm, zeros_hbm, out_hbm,
               sidx0, sidx1, didx0, didx1, rows0, rows1, acc, sem0, sem1):
        cid = lax.axis_index("c")
        sid = lax.axis_index("s")
        wid = sid * _NC + cid
        r0 = sid * rpt
        pltpu.sync_copy(zeros_hbm.at[pl.ds(r0, rpt)], acc.at[pl.ds(r0, rpt)])
        c0 = wid * per
        plsc.subcore_barrier()

        def fetch_idx(row, sbuf, dbuf):
            pltpu.sync_copy(src1_hbm.at[pl.ds(row * _CH, _CH)], sbuf)
            pltpu.sync_copy(dst1_hbm.at[pl.ds(row * _CH, _CH)], dbuf)

        def gstart(sbuf, buf, sem):
            pltpu.make_async_copy(h_hbm.at[sbuf], buf, sem).start()

        def gwait(sbuf, buf, sem):
            pltpu.make_async_copy(h_hbm.at[sbuf], buf, sem).wait()

        def scatter(dbuf, buf):
            pltpu.sync_copy(buf, acc.at[dbuf], add=True)

        def body(k, carry):
            j = c0 + k
            fetch_idx(j, sidx0, didx0)
            pltpu.async_copy(h_hbm.at[sidx0], rows0, sem0).wait()
            scatter(didx0, rows0)
            return carry

        lax.fori_loop(0, per, body, 0)

        plsc.subcore_barrier()
        pltpu.sync_copy(acc.at[pl.ds(r0, rpt)],
                        out_hbm.at[cid, pl.ds(r0, rpt)])

    return segsum


@functools.lru_cache(maxsize=None)
def _build_count(Np, per, W=128):
    NW = _NC * _NS
    rpt = Np // _NS
    mesh = plsc.VectorSubcoreMesh(core_axis_name="c", subcore_axis_name="s")

    @functools.partial(
        pl.kernel,
        mesh=mesh,
        out_type=jax.ShapeDtypeStruct((_NC, Np, W), jnp.float32),
        scratch_types=[
            pltpu.VMEM((_CH,), jnp.int32),
            pltpu.VMEM((_CH, W), jnp.float32),
            pltpu.VMEM_SHARED((Np, W), jnp.float32),
        ],
    )
    def count(dst1_hbm, ones_hbm, zerosw_hbm, out_hbm, didx, ones_v, cacc):
        cid = lax.axis_index("c")
        sid = lax.axis_index("s")
        wid = sid * _NC + cid
        r0 = sid * rpt
        pltpu.sync_copy(ones_hbm, ones_v)
        pltpu.sync_copy(zerosw_hbm.at[pl.ds(r0, rpt)], cacc.at[pl.ds(r0, rpt)])
        c0 = wid * per
        plsc.subcore_barrier()

        def body(j, carry):
            pltpu.sync_copy(dst1_hbm.at[pl.ds((c0 + j) * _CH, _CH)], didx)
            pltpu.sync_copy(ones_v, cacc.at[didx], add=True)
            return carry

        lax.fori_loop(0, per, body, 0)

        plsc.subcore_barrier()
        pltpu.sync_copy(cacc.at[pl.ds(r0, rpt)],
                        out_hbm.at[cid, pl.ds(r0, rpt)])

    return count


def _fuse(p, h, cnt, Wl_i, Wr_i, b_i, relu, nb=8):
    Np, H = h.shape
    BR = Np // nb

    def body(p_ref, h_ref, cnt_ref, wl_ref, wr_ref, b_ref, o_ref):
        inv = 1.0 / jnp.maximum(cnt_ref[...], 1.0)
        agg = (p_ref[0] + p_ref[1]) * inv
        acc = jnp.dot(agg, wl_ref[...], preferred_element_type=jnp.float32)
        acc = acc + jnp.dot(h_ref[...], wr_ref[...],
                            preferred_element_type=jnp.float32)
        acc = acc + b_ref[...]
        if relu:
            acc = jnp.maximum(acc, 0.0)
        o_ref[...] = acc

    return pl.pallas_call(
        body,
        grid=(nb,),
        in_specs=[
            pl.BlockSpec((2, BR, H), lambda i: (0, i, 0)),
            pl.BlockSpec((BR, H), lambda i: (i, 0)),
            pl.BlockSpec((BR, 1), lambda i: (i, 0)),
            pl.BlockSpec((H, H), lambda i: (0, 0)),
            pl.BlockSpec((H, H), lambda i: (0, 0)),
            pl.BlockSpec((1, H), lambda i: (0, 0)),
        ],
        out_specs=pl.BlockSpec((BR, H), lambda i: (i, 0)),
        out_shape=jax.ShapeDtypeStruct((Np, H), jnp.float32),
    )(p, h, cnt, Wl_i, Wr_i, b_i.reshape(1, H))


def kernel(x, edge_index, Wl, Wr, b):
    N, D = x.shape
    E = edge_index.shape[1]
    L = Wl.shape[0]
    NW = _NC * _NS

    Np = (N // 128 + 1) * 128
    nchunks = -(-E // _CH)
    per = -(-nchunks // NW)
    per = per + (per % 2)
    Ep = per * NW * _CH

    iota_e = jnp.arange(Ep, dtype=jnp.int32)
    pad_dst = N + iota_e % (Np - N)
    src = (iota_e % Np).at[:E].set(edge_index[0])
    dst = pad_dst.at[:E].set(edge_index[1])
    hp = jnp.zeros((Np, D), jnp.float32).at[:N].set(x)
    zeros = jnp.zeros((Np, D), jnp.float32)
    onesw = jnp.ones((_CH, D), jnp.float32)

    cparts = _build_count(Np, per, D)(dst, onesw, zeros)
    cnt = (cparts[0, :, :1] + cparts[1, :, :1])

    segsum = _build_segsum(Np, D, per)
    h = hp
    for i in range(L):
        p = segsum(h, src, dst, zeros)
        h = _fuse(p, h, cnt, Wl[i], Wr[i], b[i], relu=(i < L - 1))
    return h[:N]


# speedup vs baseline: 3.4252x; 1.5367x over previous
import functools

import jax
import jax.numpy as jnp
from jax import lax
from jax.experimental import pallas as pl
from jax.experimental.pallas import tpu as pltpu
from jax.experimental.pallas import tpu_sc as plsc

_NC = 2
_NS = 16
_CH = 128


@functools.lru_cache(maxsize=None)
def _build_segsum(Np, H, per):
    NW = _NC * _NS
    rpt = Np // _NS
    mesh = plsc.VectorSubcoreMesh(core_axis_name="c", subcore_axis_name="s")

    @functools.partial(
        pl.kernel,
        mesh=mesh,
        out_type=jax.ShapeDtypeStruct((_NC, Np, H), jnp.float32),
        scratch_types=[
            pltpu.VMEM((_CH,), jnp.int32),
            pltpu.VMEM((_CH,), jnp.int32),
            pltpu.VMEM((_CH,), jnp.int32),
            pltpu.VMEM((_CH,), jnp.int32),
            pltpu.VMEM((_CH, H), jnp.float32),
            pltpu.VMEM((_CH, H), jnp.float32),
            pltpu.VMEM_SHARED((Np, H), jnp.float32),
            pltpu.SemaphoreType.DMA,
            pltpu.SemaphoreType.DMA,
        ],
    )
    def segsum(h_hbm, src1_hbm, dst1_hbm, zeros_hbm, out_hbm,
               sidx0, sidx1, didx0, didx1, rows0, rows1, acc, sem0, sem1):
        cid = lax.axis_index("c")
        sid = lax.axis_index("s")
        wid = sid * _NC + cid
        r0 = sid * rpt
        pltpu.sync_copy(zeros_hbm.at[pl.ds(r0, rpt)], acc.at[pl.ds(r0, rpt)])
        c0 = wid * per
        plsc.subcore_barrier()

        def fetch_idx(row, sbuf, dbuf):
            pltpu.sync_copy(src1_hbm.at[pl.ds(row * _CH, _CH)], sbuf)
            pltpu.sync_copy(dst1_hbm.at[pl.ds(row * _CH, _CH)], dbuf)

        def gstart(sbuf, buf, sem):
            pltpu.make_async_copy(h_hbm.at[sbuf], buf, sem).start()

        def gwait(sbuf, buf, sem):
            pltpu.make_async_copy(h_hbm.at[sbuf], buf, sem).wait()

        def scatter(dbuf, buf):
            pltpu.sync_copy(buf, acc.at[dbuf], add=True)

        fetch_idx(c0, sidx0, didx0)
        gstart(sidx0, rows0, sem0)

        def body(k, carry):
            j = c0 + 2 * k
            fetch_idx(j + 1, sidx1, didx1)
            gstart(sidx1, rows1, sem1)
            gwait(sidx0, rows0, sem0)
            scatter(didx0, rows0)

            @pl.when(2 * k + 2 < per)
            def _():
                fetch_idx(j + 2, sidx0, didx0)
                gstart(sidx0, rows0, sem0)

            gwait(sidx1, rows1, sem1)
            scatter(didx1, rows1)
            return carry

        lax.fori_loop(0, per // 2, body, 0)

        plsc.subcore_barrier()
        pltpu.sync_copy(acc.at[pl.ds(r0, rpt)],
                        out_hbm.at[cid, pl.ds(r0, rpt)])

    return segsum


@functools.lru_cache(maxsize=None)
def _build_count(Np, per, W=128):
    NW = _NC * _NS
    rpt = Np // _NS
    mesh = plsc.VectorSubcoreMesh(core_axis_name="c", subcore_axis_name="s")

    @functools.partial(
        pl.kernel,
        mesh=mesh,
        out_type=jax.ShapeDtypeStruct((_NC, Np, W), jnp.float32),
        scratch_types=[
            pltpu.VMEM((_CH,), jnp.int32),
            pltpu.VMEM((_CH, W), jnp.float32),
            pltpu.VMEM_SHARED((Np, W), jnp.float32),
        ],
    )
    def count(dst1_hbm, ones_hbm, zerosw_hbm, out_hbm, didx, ones_v, cacc):
        cid = lax.axis_index("c")
        sid = lax.axis_index("s")
        wid = sid * _NC + cid
        r0 = sid * rpt
        pltpu.sync_copy(ones_hbm, ones_v)
        pltpu.sync_copy(zerosw_hbm.at[pl.ds(r0, rpt)], cacc.at[pl.ds(r0, rpt)])
        c0 = wid * per
        plsc.subcore_barrier()

        def body(j, carry):
            pltpu.sync_copy(dst1_hbm.at[pl.ds((c0 + j) * _CH, _CH)], didx)
            pltpu.sync_copy(ones_v, cacc.at[didx], add=True)
            return carry

        lax.fori_loop(0, per, body, 0)

        plsc.subcore_barrier()
        pltpu.sync_copy(cacc.at[pl.ds(r0, rpt)],
                        out_hbm.at[cid, pl.ds(r0, rpt)])

    return count


def _fuse(p, h, cnt, Wl_i, Wr_i, b_i, relu, nb=8):
    Np, H = h.shape
    BR = Np // nb

    def body(p_ref, h_ref, cnt_ref, wl_ref, wr_ref, b_ref, o_ref):
        inv = 1.0 / jnp.maximum(cnt_ref[...], 1.0)
        agg = (p_ref[0] + p_ref[1]) * inv
        acc = jnp.dot(agg, wl_ref[...], preferred_element_type=jnp.float32)
        acc = acc + jnp.dot(h_ref[...], wr_ref[...],
                            preferred_element_type=jnp.float32)
        acc = acc + b_ref[...]
        if relu:
            acc = jnp.maximum(acc, 0.0)
        o_ref[...] = acc

    return pl.pallas_call(
        body,
        grid=(nb,),
        in_specs=[
            pl.BlockSpec((2, BR, H), lambda i: (0, i, 0)),
            pl.BlockSpec((BR, H), lambda i: (i, 0)),
            pl.BlockSpec((BR, 1), lambda i: (i, 0)),
            pl.BlockSpec((H, H), lambda i: (0, 0)),
            pl.BlockSpec((H, H), lambda i: (0, 0)),
            pl.BlockSpec((1, H), lambda i: (0, 0)),
        ],
        out_specs=pl.BlockSpec((BR, H), lambda i: (i, 0)),
        out_shape=jax.ShapeDtypeStruct((Np, H), jnp.float32),
    )(p, h, cnt, Wl_i, Wr_i, b_i.reshape(1, H))


def kernel(x, edge_index, Wl, Wr, b):
    N, D = x.shape
    E = edge_index.shape[1]
    L = Wl.shape[0]
    NW = _NC * _NS

    Np = (N // 128 + 1) * 128
    nchunks = -(-E // _CH)
    per = -(-nchunks // NW)
    per = per + (per % 2)
    Ep = per * NW * _CH

    iota_e = jnp.arange(Ep, dtype=jnp.int32)
    pad_dst = N + iota_e % (Np - N)
    src = (iota_e % Np).at[:E].set(edge_index[0])
    dst = pad_dst.at[:E].set(edge_index[1])
    hp = jnp.zeros((Np, D), jnp.float32).at[:N].set(x)
    zeros = jnp.zeros((Np, D), jnp.float32)
    onesw = jnp.ones((_CH, D), jnp.float32)

    cparts = _build_count(Np, per, D)(dst, onesw, zeros)
    cnt = (cparts[0, :, :1] + cparts[1, :, :1])

    segsum = _build_segsum(Np, D, per)
    h = hp
    for i in range(L):
        p = segsum(h, src, dst, zeros)
        h = _fuse(p, h, cnt, Wl[i], Wr[i], b[i], relu=(i < L - 1))
    return h[:N]


# speedup vs baseline: 3.7397x; 1.0918x over previous
import functools

import jax
import jax.numpy as jnp
from jax import lax
from jax.experimental import pallas as pl
from jax.experimental.pallas import tpu as pltpu
from jax.experimental.pallas import tpu_sc as plsc

_NC = 2
_NS = 16
_CH = 128
_NB = 3


@functools.lru_cache(maxsize=None)
def _build_segsum(Np, H, per):
    NW = _NC * _NS
    rpt = Np // _NS
    assert per % _NB == 0
    mesh = plsc.VectorSubcoreMesh(core_axis_name="c", subcore_axis_name="s")

    @functools.partial(
        pl.kernel,
        mesh=mesh,
        out_type=jax.ShapeDtypeStruct((_NC, Np, H), jnp.float32),
        scratch_types=[
            pltpu.VMEM((_CH,), jnp.int32),
            pltpu.VMEM((_CH,), jnp.int32),
            pltpu.VMEM((_CH,), jnp.int32),
            pltpu.VMEM((_CH,), jnp.int32),
            pltpu.VMEM((_CH,), jnp.int32),
            pltpu.VMEM((_CH,), jnp.int32),
            pltpu.VMEM((_CH, H), jnp.float32),
            pltpu.VMEM((_CH, H), jnp.float32),
            pltpu.VMEM((_CH, H), jnp.float32),
            pltpu.VMEM_SHARED((Np, H), jnp.float32),
            pltpu.SemaphoreType.DMA,
            pltpu.SemaphoreType.DMA,
            pltpu.SemaphoreType.DMA,
            pltpu.SemaphoreType.DMA,
            pltpu.SemaphoreType.DMA,
            pltpu.SemaphoreType.DMA,
        ],
    )
    def segsum(h_hbm, src_hbm, dst_hbm, zeros_hbm, out_hbm,
               sidx0, sidx1, sidx2, didx0, didx1, didx2,
               rows0, rows1, rows2, acc,
               gsem0, gsem1, gsem2, ssem0, ssem1, ssem2):
        cid = lax.axis_index("c")
        sid = lax.axis_index("s")
        wid = sid * _NC + cid
        r0 = sid * rpt
        pltpu.sync_copy(zeros_hbm.at[pl.ds(r0, rpt)], acc.at[pl.ds(r0, rpt)])
        c0 = wid * per
        plsc.subcore_barrier()

        sets = (
            (sidx0, didx0, rows0, gsem0, ssem0),
            (sidx1, didx1, rows1, gsem1, ssem1),
            (sidx2, didx2, rows2, gsem2, ssem2),
        )

        def launch(t, sb, db, rb, gs, ss, k):
            @pl.when(k > 0)
            def _():
                pltpu.make_async_copy(rb, acc.at[db], ss).wait()
            pltpu.sync_copy(src_hbm.at[pl.ds(t * _CH, _CH)], sb)
            pltpu.sync_copy(dst_hbm.at[pl.ds(t * _CH, _CH)], db)
            pltpu.make_async_copy(h_hbm.at[sb], rb, gs).start()

        def body(k, carry):
            q = c0 + _NB * k
            for i, (sb, db, rb, gs, ss) in enumerate(sets):
                launch(q + i, sb, db, rb, gs, ss, k)
            for i, (sb, db, rb, gs, ss) in enumerate(sets):
                pltpu.make_async_copy(h_hbm.at[sb], rb, gs).wait()
                pltpu.make_async_copy(rb, acc.at[db], ss).start(add=True)
            return carry

        lax.fori_loop(0, per // _NB, body, 0)
        for (sb, db, rb, gs, ss) in sets:
            pltpu.make_async_copy(rb, acc.at[db], ss).wait()

        plsc.subcore_barrier()
        pltpu.sync_copy(acc.at[pl.ds(r0, rpt)],
                        out_hbm.at[cid, pl.ds(r0, rpt)])

    return segsum


@functools.lru_cache(maxsize=None)
def _build_count(Np, per, W=128):
    NW = _NC * _NS
    rpt = Np // _NS
    assert per % _NB == 0
    mesh = plsc.VectorSubcoreMesh(core_axis_name="c", subcore_axis_name="s")

    @functools.partial(
        pl.kernel,
        mesh=mesh,
        out_type=jax.ShapeDtypeStruct((_NC, Np, W), jnp.float32),
        scratch_types=[
            pltpu.VMEM((_CH,), jnp.int32),
            pltpu.VMEM((_CH,), jnp.int32),
            pltpu.VMEM((_CH,), jnp.int32),
            pltpu.VMEM((_CH, W), jnp.float32),
            pltpu.VMEM_SHARED((Np, W), jnp.float32),
            pltpu.SemaphoreType.DMA,
            pltpu.SemaphoreType.DMA,
            pltpu.SemaphoreType.DMA,
        ],
    )
    def count(dst_hbm, ones_hbm, zerosw_hbm, out_hbm,
              didx0, didx1, didx2, ones_v, cacc, ssem0, ssem1, ssem2):
        cid = lax.axis_index("c")
        sid = lax.axis_index("s")
        wid = sid * _NC + cid
        r0 = sid * rpt
        pltpu.sync_copy(ones_hbm, ones_v)
        pltpu.sync_copy(zerosw_hbm.at[pl.ds(r0, rpt)], cacc.at[pl.ds(r0, rpt)])
        c0 = wid * per
        plsc.subcore_barrier()

        sets = ((didx0, ssem0), (didx1, ssem1), (didx2, ssem2))

        def body(k, carry):
            q = c0 + _NB * k
            for i, (db, ss) in enumerate(sets):
                @pl.when(k > 0)
                def _():
                    pltpu.make_async_copy(ones_v, cacc.at[db], ss).wait()
                pltpu.sync_copy(dst_hbm.at[pl.ds((q + i) * _CH, _CH)], db)
                pltpu.make_async_copy(ones_v, cacc.at[db], ss).start(add=True)
            return carry

        lax.fori_loop(0, per // _NB, body, 0)
        for (db, ss) in sets:
            pltpu.make_async_copy(ones_v, cacc.at[db], ss).wait()

        plsc.subcore_barrier()
        pltpu.sync_copy(cacc.at[pl.ds(r0, rpt)],
                        out_hbm.at[cid, pl.ds(r0, rpt)])

    return count


def _fuse(p, h, cnt, Wl_i, Wr_i, b_i, relu, nb=8):
    Np, H = h.shape
    BR = Np // nb

    def body(p_ref, h_ref, cnt_ref, wl_ref, wr_ref, b_ref, o_ref):
        inv = 1.0 / jnp.maximum(cnt_ref[...], 1.0)
        agg = (p_ref[0] + p_ref[1]) * inv
        acc = jnp.dot(agg, wl_ref[...], preferred_element_type=jnp.float32)
        acc = acc + jnp.dot(h_ref[...], wr_ref[...],
                            preferred_element_type=jnp.float32)
        acc = acc + b_ref[...]
        if relu:
            acc = jnp.maximum(acc, 0.0)
        o_ref[...] = acc

    return pl.pallas_call(
        body,
        grid=(nb,),
        in_specs=[
            pl.BlockSpec((2, BR, H), lambda i: (0, i, 0)),
            pl.BlockSpec((BR, H), lambda i: (i, 0)),
            pl.BlockSpec((BR, 1), lambda i: (i, 0)),
            pl.BlockSpec((H, H), lambda i: (0, 0)),
            pl.BlockSpec((H, H), lambda i: (0, 0)),
            pl.BlockSpec((1, H), lambda i: (0, 0)),
        ],
        out_specs=pl.BlockSpec((BR, H), lambda i: (i, 0)),
        out_shape=jax.ShapeDtypeStruct((Np, H), jnp.float32),
    )(p, h, cnt, Wl_i, Wr_i, b_i.reshape(1, H))


def kernel(x, edge_index, Wl, Wr, b):
    N, D = x.shape
    E = edge_index.shape[1]
    L = Wl.shape[0]
    NW = _NC * _NS

    Np = (N // 128 + 1) * 128
    nchunks = -(-E // _CH)
    per = -(-nchunks // NW)
    per = per + (-per) % _NB
    Ep = per * NW * _CH

    iota_e = jnp.arange(Ep, dtype=jnp.int32)
    pad_dst = N + iota_e % (Np - N)
    src = (iota_e % Np).at[:E].set(edge_index[0])
    dst = pad_dst.at[:E].set(edge_index[1])
    hp = jnp.zeros((Np, D), jnp.float32).at[:N].set(x)
    zeros = jnp.zeros((Np, D), jnp.float32)
    onesw = jnp.ones((_CH, D), jnp.float32)

    cparts = _build_count(Np, per, D)(dst, onesw, zeros)
    cnt = (cparts[0, :, :1] + cparts[1, :, :1])

    segsum = _build_segsum(Np, D, per)
    h = hp
    for i in range(L):
        p = segsum(h, src, dst, zeros)
        h = _fuse(p, h, cnt, Wl[i], Wr[i], b[i], relu=(i < L - 1))
    return h[:N]
